# baseline (speedup 1.0000x reference)
import functools

import jax
import jax.numpy as jnp
from jax.experimental import pallas as pl

N_TOKENS = 32768
N_EMBED = 768
NUM_EXPERTS = 8
TOP_K = 2

BT = 2048


def _router_kernel(x_ref, wt_ref, b_ref, outt_ref, idxt_ref):
    logits = jax.lax.dot_general(
        x_ref[...], wt_ref[...],
        dimension_numbers=(((1,), (0,)), ((), ())),
        preferred_element_type=jnp.float32,
    )
    lt = logits.T + b_ref[...]

    se = jax.lax.broadcasted_iota(jnp.int32, lt.shape, 0).astype(jnp.float32)
    m1 = jnp.max(lt, axis=0, keepdims=True)
    i1 = jnp.min(jnp.where(lt == m1, se, 8.0), axis=0, keepdims=True)
    masked = jnp.where(se == i1, -jnp.inf, lt)
    m2 = jnp.max(masked, axis=0, keepdims=True)
    i2 = jnp.min(jnp.where(masked == m2, se, 8.0), axis=0, keepdims=True)

    e2 = jnp.exp(m2 - m1)
    p1 = 1.0 / (1.0 + e2)
    p2 = e2 * p1

    outt_ref[...] = jnp.where(se == i1, p1, jnp.where(se == i2, p2, 0.0))
    idxt_ref[...] = jnp.concatenate([i1, i2], axis=0).astype(jnp.int32)


@functools.partial(jax.jit, static_argnames=())
def kernel(x, W, b):
    n_tokens = x.shape[0]
    grid = (n_tokens // BT,)
    wt = W.T
    b2 = b.reshape(NUM_EXPERTS, 1)
    outt, idxt = pl.pallas_call(
        _router_kernel,
        grid=grid,
        in_specs=[
            pl.BlockSpec((BT, N_EMBED), lambda i: (i, 0)),
            pl.BlockSpec((N_EMBED, NUM_EXPERTS), lambda i: (0, 0)),
            pl.BlockSpec((NUM_EXPERTS, 1), lambda i: (0, 0)),
        ],
        out_specs=[
            pl.BlockSpec((NUM_EXPERTS, BT), lambda i: (0, i)),
            pl.BlockSpec((TOP_K, BT), lambda i: (0, i)),
        ],
        out_shape=[
            jax.ShapeDtypeStruct((NUM_EXPERTS, n_tokens), jnp.float32),
            jax.ShapeDtypeStruct((TOP_K, n_tokens), jnp.int32),
        ],
    )(x, wt, b2)
    return outt.T, idxt.T
